# Initial kernel scaffold; baseline (speedup 1.0000x reference)
import jax
import jax.numpy as jnp
from jax.experimental import pallas as pl


def kernel(node_embed, segment_ids, start_idx, end_idx):
    raise NotImplementedError("write your pallas kernel here")


# speedup vs baseline: 2.7482x; 2.7482x over previous
import jax
import jax.numpy as jnp
from jax import lax
from jax.experimental import pallas as pl
from jax.experimental.pallas import tpu as pltpu
from jax.experimental.pallas import tpu_sc as plsc

N = 100000
D = 128
B = 8192
NC = 2
NS = 16
HC = D // NC
GPW = B // (NC * NS)
RPS = 6272
TAIL_FULL = 46
TAIL_REM = N - 15 * RPS - TAIL_FULL * 128


def _body(ne, sid, st, en, out, gidx, grows, ids2, rows, stage, idt, rowst,
          acc, sem):
    c = lax.axis_index("c")
    s = lax.axis_index("s")
    w = c * NS + s
    col0 = pl.multiple_of(c * HC, HC)

    base = w * GPW
    for src, ocol in ((st, D), (en, 2 * D)):
        for h in range(2):
            pltpu.sync_copy(src.at[pl.ds(base + h * 128, 128)], gidx.at[h])
        for h in range(2):
            pltpu.async_copy(ne.at[gidx.at[h]], grows.at[pl.ds(h * 128, 128)],
                             sem).wait()
        pltpu.sync_copy(grows, out.at[pl.ds(base, GPW), pl.ds(ocol, D)])

    def zero_step(i, _):
        r = i // (HC // 16)
        j = i % (HC // 16)
        stage[r, pl.ds(j * 16, 16)] = jnp.zeros((16,), jnp.float32)
        return 0
    lax.fori_loop(0, 512 * (HC // 16), zero_step, 0)
    pltpu.sync_copy(stage, acc.at[pl.ds(s * 512, 512)])
    plsc.subcore_barrier()

    r0 = s * RPS

    def acc_chunk(k, _):
        start = r0 + k * 128
        pltpu.sync_copy(sid.at[pl.ds(start, 128)], ids2.at[0])
        pltpu.sync_copy(ne.at[pl.ds(start, 128), pl.ds(col0, HC)], rows)
        pltpu.sync_copy(rows, acc.at[ids2.at[0]], add=True)
        return 0

    lax.fori_loop(0, TAIL_FULL, acc_chunk, 0)

    @pl.when(s < NS - 1)
    def _more():
        lax.fori_loop(TAIL_FULL, RPS // 128, acc_chunk, 0)

    @pl.when(s == NS - 1)
    def _tail():
        start = r0 + TAIL_FULL * 128
        pltpu.sync_copy(sid.at[pl.ds(start, TAIL_REM)], idt.at[0])
        pltpu.sync_copy(ne.at[pl.ds(start, TAIL_REM), pl.ds(col0, HC)], rowst)
        pltpu.sync_copy(rowst, acc.at[idt.at[0]], add=True)

    plsc.subcore_barrier()

    pltpu.sync_copy(acc.at[pl.ds(s * 512, 512)], stage)
    pltpu.sync_copy(stage, out.at[pl.ds(s * 512, 512), pl.ds(col0, HC)])


_sc_call = pl.kernel(
    _body,
    out_type=jax.ShapeDtypeStruct((B, 3 * D), jnp.float32),
    mesh=plsc.VectorSubcoreMesh(core_axis_name="c", subcore_axis_name="s"),
    scratch_types=[
        pltpu.VMEM((2, 128), jnp.int32),
        pltpu.VMEM((GPW, D), jnp.float32),
        pltpu.VMEM((2, 128), jnp.int32),
        pltpu.VMEM((128, HC), jnp.float32),
        pltpu.VMEM((512, HC), jnp.float32),
        pltpu.VMEM((1, TAIL_REM), jnp.int32),
        pltpu.VMEM((TAIL_REM, HC), jnp.float32),
        pltpu.VMEM_SHARED((B, HC), jnp.float32),
        pltpu.SemaphoreType.DMA,
    ],
    compiler_params=pltpu.CompilerParams(use_tc_tiling_on_sc=False),
)


def kernel(node_embed, segment_ids, start_idx, end_idx):
    return _sc_call(node_embed,
                    segment_ids.astype(jnp.int32),
                    start_idx.astype(jnp.int32),
                    end_idx.astype(jnp.int32))


# speedup vs baseline: 5.0308x; 1.8306x over previous
import jax
import jax.numpy as jnp
from jax import lax
from jax.experimental import pallas as pl
from jax.experimental.pallas import tpu as pltpu
from jax.experimental.pallas import tpu_sc as plsc

N = 100000
D = 128
B = 8192
NC = 2
NS = 16
HC = D // NC
GPW = B // (NC * NS)
RPS = 6272
NFA = RPS // 128
NFB = 46
TAIL_REM = N - 15 * RPS - NFB * 128


def _body(ne, sid, st, en, out, gidx, grows, ids, rows, idt, rowst,
          acc, gsem, isem, rsem):
    c = lax.axis_index("c")
    s = lax.axis_index("s")
    w = c * NS + s
    col0 = pl.multiple_of(c * HC, HC)

    base = w * GPW
    for h in range(2):
        pltpu.async_copy(st.at[pl.ds(base + h * 128, 128)], gidx.at[h], isem)
        pltpu.async_copy(en.at[pl.ds(base + h * 128, 128)], gidx.at[2 + h],
                         isem)
    for h in range(4):
        pltpu.make_async_copy(st.at[pl.ds(0, 128)], gidx.at[0], isem).wait()
    for h in range(4):
        pltpu.async_copy(ne.at[gidx.at[h]], grows.at[pl.ds(h * 128, 128)],
                         gsem)

    def zero_step(i, _):
        r = i // (HC // 16)
        j = i % (HC // 16)
        rows[0, r, pl.ds(j * 16, 16)] = jnp.zeros((16,), jnp.float32)
        return 0
    lax.fori_loop(0, 128 * (HC // 16), zero_step, 0)
    for k in range(4):
        pltpu.sync_copy(rows.at[0], acc.at[pl.ds(s * 512 + k * 128, 128)])
    plsc.subcore_barrier()

    r0 = s * RPS
    nf = jnp.where(s < NS - 1, NFA, NFB)

    def fire(g, slot):
        start = r0 + g * 128
        pltpu.async_copy(sid.at[pl.ds(start, 128)], ids.at[slot, 0], isem)
        pltpu.async_copy(ne.at[pl.ds(start, 128), pl.ds(col0, HC)],
                         rows.at[slot], rsem)

    fire(0, 0)

    def it(g, _):
        @pl.when(g + 1 < nf)
        def _prefetch():
            fire(g + 1, lax.rem(g + 1, 2))
        pltpu.make_async_copy(sid.at[pl.ds(0, 128)], ids.at[0, 0],
                              isem).wait()
        pltpu.make_async_copy(ne.at[pl.ds(0, 128), pl.ds(col0, HC)],
                              rows.at[0], rsem).wait()
        slot = lax.rem(g, 2)
        pltpu.sync_copy(rows.at[slot], acc.at[ids.at[slot, 0]], add=True)
        return 0

    lax.fori_loop(0, nf, it, 0)

    @pl.when(s == NS - 1)
    def _tail():
        start = r0 + NFB * 128
        pltpu.sync_copy(sid.at[pl.ds(start, TAIL_REM)], idt.at[0])
        pltpu.sync_copy(ne.at[pl.ds(start, TAIL_REM), pl.ds(col0, HC)], rowst)
        pltpu.sync_copy(rowst, acc.at[idt.at[0]], add=True)

    plsc.subcore_barrier()

    for h in range(4):
        pltpu.make_async_copy(ne.at[gidx.at[0]], grows.at[pl.ds(0, 128)],
                              gsem).wait()
    pltpu.sync_copy(grows.at[pl.ds(0, GPW)], out.at[pl.ds(base, GPW),
                                                    pl.ds(D, D)])
    pltpu.sync_copy(grows.at[pl.ds(GPW, GPW)], out.at[pl.ds(base, GPW),
                                                      pl.ds(2 * D, D)])

    for k in range(4):
        slot = k % 2
        pltpu.sync_copy(acc.at[pl.ds(s * 512 + k * 128, 128)], rows.at[slot])
        pltpu.sync_copy(rows.at[slot],
                        out.at[pl.ds(s * 512 + k * 128, 128),
                               pl.ds(col0, HC)])


_sc_call = pl.kernel(
    _body,
    out_type=jax.ShapeDtypeStruct((B, 3 * D), jnp.float32),
    mesh=plsc.VectorSubcoreMesh(core_axis_name="c", subcore_axis_name="s"),
    scratch_types=[
        pltpu.VMEM((4, 128), jnp.int32),
        pltpu.VMEM((2 * GPW, D), jnp.float32),
        pltpu.VMEM((2, 1, 128), jnp.int32),
        pltpu.VMEM((2, 128, HC), jnp.float32),
        pltpu.VMEM((1, TAIL_REM), jnp.int32),
        pltpu.VMEM((TAIL_REM, HC), jnp.float32),
        pltpu.VMEM_SHARED((B, HC), jnp.float32),
        pltpu.SemaphoreType.DMA,
        pltpu.SemaphoreType.DMA,
        pltpu.SemaphoreType.DMA,
    ],
    compiler_params=pltpu.CompilerParams(use_tc_tiling_on_sc=False),
)


def kernel(node_embed, segment_ids, start_idx, end_idx):
    return _sc_call(node_embed,
                    segment_ids.astype(jnp.int32),
                    start_idx.astype(jnp.int32),
                    end_idx.astype(jnp.int32))


# speedup vs baseline: 5.3621x; 1.0659x over previous
import jax
import jax.numpy as jnp
from jax import lax
from jax.experimental import pallas as pl
from jax.experimental.pallas import tpu as pltpu
from jax.experimental.pallas import tpu_sc as plsc

N = 100000
D = 128
B = 8192
NC = 2
NS = 16
CUT = B // 2
GPW = B // (NC * NS)
CH = 64
NCH = N // CH
TAIL = N - NCH * CH
SPS = 6272


def _body(ne, sid, sid3, st, en, out, gidx, grows, ids, rows, sbuf, idt,
          rowst, cnt_v, cmat, acc, cnt_sh, gsem, isem, rsem):
    c = lax.axis_index("c")
    s = lax.axis_index("s")
    w = c * NS + s

    base = pl.multiple_of(w * GPW, GPW)
    for h in range(2):
        pltpu.async_copy(st.at[pl.ds(base + h * 128, 128)], gidx.at[h], isem)
        pltpu.async_copy(en.at[pl.ds(base + h * 128, 128)], gidx.at[2 + h],
                         isem)
    for _ in range(4):
        pltpu.make_async_copy(st.at[pl.ds(0, 128)], gidx.at[0], isem).wait()
    for h in range(2):
        pltpu.async_copy(ne.at[gidx.at[h]], grows.at[pl.ds(h * 128, 128)],
                         gsem)

    def zero_step(i, _):
        rows[0, i // 8, pl.ds((i % 8) * 16, 16)] = jnp.zeros((16,),
                                                             jnp.float32)
        return 0
    lax.fori_loop(0, CH * 8, zero_step, 0)
    zbase = pl.multiple_of(s * 512, 512)
    for k in range(8):
        pltpu.sync_copy(rows.at[0], acc.at[pl.ds(zbase + k * CH, CH)])

    soff = pl.multiple_of(s * SPS, 8)

    def count_span(off, sz, cvec):
        pltpu.sync_copy(sid.at[pl.ds(off, sz)], sbuf.at[pl.ds(0, sz)])

        def ld(i, cv):
            v = sbuf[pl.ds(i * 16, 16)]
            return cv + jnp.where(v < CUT, 1, 0).astype(jnp.int32)
        return lax.fori_loop(0, sz // 16, ld, cvec)

    cvec = jnp.zeros((16,), jnp.int32)
    for f in range(5):
        cvec = count_span(soff + f * 1024, 1024, cvec)

    @pl.when(s < NS - 1)
    def _scan_a():
        cv = count_span(soff + 5 * 1024, 1024, cvec)
        cv = count_span(soff + 6 * 1024, 128, cv)
        for j in range(8):
            cnt_v[pl.ds(16 * j, 16)] = cv

    @pl.when(s == NS - 1)
    def _scan_b():
        cv = count_span(soff + 5 * 1024, 768, cvec)
        cv = count_span(soff + 5 * 1024 + 768, TAIL, cv)
        for j in range(8):
            cnt_v[pl.ds(16 * j, 16)] = cv

    pltpu.sync_copy(cnt_v, cnt_sh.at[s])
    plsc.subcore_barrier()
    pltpu.sync_copy(cnt_sh, cmat)

    def addrow(i, tv):
        return tv + cmat[i, pl.ds(0, 16)]
    tot = lax.fori_loop(0, NS, addrow, jnp.zeros((16,), jnp.int32))
    r_star = jnp.sum(tot)

    for _ in range(2):
        pltpu.make_async_copy(ne.at[gidx.at[0]], grows.at[pl.ds(0, 128)],
                              gsem).wait()
    pltpu.sync_copy(grows, out.at[pl.ds(base, GPW), pl.ds(D, D)])
    for h in range(2):
        pltpu.async_copy(ne.at[gidx.at[2 + h]], grows.at[pl.ds(h * 128, 128)],
                         gsem)

    nc0 = jnp.minimum((r_star + CH - 1) // CH, NCH)
    k0 = jnp.minimum(r_star // CH, NCH)
    my_lo = jnp.where(c == 0, s, k0 + s)
    my_hi = jnp.where(c == 0, nc0, NCH)
    nit = jnp.maximum(my_hi - my_lo + NS - 1, 0) // NS

    def fire(i, slot):
        k = my_lo + i * NS
        start = pl.multiple_of(k * CH, CH)
        pltpu.async_copy(sid3.at[k // 2, lax.rem(k, 2)], ids.at[slot, 0],
                         isem)
        pltpu.async_copy(ne.at[pl.ds(start, CH)], rows.at[slot], rsem)

    @pl.when(nit > 0)
    def _prime():
        fire(0, 0)

    def it(g, _):
        @pl.when(g + 1 < nit)
        def _prefetch():
            fire(g + 1, lax.rem(g + 1, 2))
        pltpu.make_async_copy(sid3.at[0, 0], ids.at[0, 0], isem).wait()
        pltpu.make_async_copy(ne.at[pl.ds(0, CH)], rows.at[0], rsem).wait()
        slot = lax.rem(g, 2)
        pltpu.sync_copy(rows.at[slot], acc.at[ids.at[slot, 0]], add=True)
        return 0

    lax.fori_loop(0, nit, it, 0)

    @pl.when(s == NS - 1)
    def _tail():
        pltpu.sync_copy(sid.at[pl.ds(NCH * CH, TAIL)], idt.at[0])
        pltpu.sync_copy(ne.at[pl.ds(NCH * CH, TAIL)], rowst)
        pltpu.sync_copy(rowst, acc.at[idt.at[0]], add=True)

    for _ in range(2):
        pltpu.make_async_copy(ne.at[gidx.at[0]], grows.at[pl.ds(0, 128)],
                              gsem).wait()
    pltpu.sync_copy(grows, out.at[pl.ds(base, GPW), pl.ds(2 * D, D)])

    plsc.subcore_barrier()

    obase = pl.multiple_of(c * CUT + s * GPW, GPW)
    pltpu.sync_copy(acc.at[pl.ds(obase, GPW)], grows)
    pltpu.sync_copy(grows, out.at[pl.ds(obase, GPW), pl.ds(0, D)])


_sc_call = pl.kernel(
    _body,
    out_type=jax.ShapeDtypeStruct((B, 3 * D), jnp.float32),
    mesh=plsc.VectorSubcoreMesh(core_axis_name="c", subcore_axis_name="s"),
    scratch_types=[
        pltpu.VMEM((4, 128), jnp.int32),
        pltpu.VMEM((GPW, D), jnp.float32),
        pltpu.VMEM((2, 1, CH), jnp.int32),
        pltpu.VMEM((2, CH, D), jnp.float32),
        pltpu.VMEM((1024,), jnp.int32),
        pltpu.VMEM((1, TAIL), jnp.int32),
        pltpu.VMEM((TAIL, D), jnp.float32),
        pltpu.VMEM((128,), jnp.int32),
        pltpu.VMEM((NS, 128), jnp.int32),
        pltpu.VMEM_SHARED((B, D), jnp.float32),
        pltpu.VMEM_SHARED((NS, 128), jnp.int32),
        pltpu.SemaphoreType.DMA,
        pltpu.SemaphoreType.DMA,
        pltpu.SemaphoreType.DMA,
    ],
    compiler_params=pltpu.CompilerParams(use_tc_tiling_on_sc=True,
                                         needs_layout_passes=False),
)


def kernel(node_embed, segment_ids, start_idx, end_idx):
    sid = segment_ids.astype(jnp.int32)
    sid3 = jnp.pad(sid, (0, 2 * CH * ((N + 2 * CH - 1) // (2 * CH)) - N)
                   ).reshape(-1, 2, CH)
    return _sc_call(node_embed, sid, sid3,
                    start_idx.astype(jnp.int32),
                    end_idx.astype(jnp.int32))


# speedup vs baseline: 6.1027x; 1.1381x over previous
import jax
import jax.numpy as jnp
from jax import lax
from jax.experimental import pallas as pl
from jax.experimental.pallas import tpu as pltpu
from jax.experimental.pallas import tpu_sc as plsc

N = 100000
D = 128
B = 8192
NC = 2
NS = 16
CUT = B // 2
GPW = B // (NC * NS)
CH = 128
NCH = N // CH
TAIL = N - NCH * CH
SPS = 6272
SPL = N - (NS - 1) * SPS


def _body(ne, sid, st, en, out, gidx, ids, rows, sbuf, idt, rowst,
          cnt_v, cmat, acc, cnt_sh, gsem, isem, rsem, psem):
    c = lax.axis_index("c")
    s = lax.axis_index("s")
    w = c * NS + s

    soff = pl.multiple_of(s * SPS, 8)

    @pl.when(s < NS - 1)
    def _ps_a():
        pltpu.async_copy(sid.at[pl.ds(soff, SPS)], sbuf, psem)

    @pl.when(s == NS - 1)
    def _ps_b():
        pltpu.async_copy(sid.at[pl.ds(soff, SPL)], sbuf.at[pl.ds(0, SPL)],
                         psem)
        pltpu.async_copy(sid.at[pl.ds(NCH * CH, TAIL)],
                         sbuf.at[pl.ds(SPL, TAIL)], psem)

    base = pl.multiple_of(w * GPW, GPW)
    for h in range(2):
        pltpu.async_copy(st.at[pl.ds(base + h * 128, 128)], gidx.at[h], isem)
        pltpu.async_copy(en.at[pl.ds(base + h * 128, 128)], gidx.at[2 + h],
                         isem)
    for _ in range(4):
        pltpu.make_async_copy(st.at[pl.ds(0, 128)], gidx.at[0], isem).wait()
    for h in range(2):
        pltpu.async_copy(ne.at[gidx.at[h]], rows.at[h], gsem)

    def zero_step(i, _):
        rowst[i // 8, pl.ds((i % 8) * 16, 16)] = jnp.zeros((16,), jnp.float32)
        return 0
    lax.fori_loop(0, TAIL * 8, zero_step, 0)
    zbase = pl.multiple_of(s * 512, 512)
    for k in range(512 // TAIL):
        pltpu.sync_copy(rowst, acc.at[pl.ds(zbase + k * TAIL, TAIL)])

    def count_groups(ngroups, cvec):
        def ld(i, cv):
            v = sbuf[pl.ds(i * 16, 16)]
            return cv + jnp.where(v < CUT, 1, 0).astype(jnp.int32)
        return lax.fori_loop(0, ngroups, ld, cvec)

    cvec = jnp.zeros((16,), jnp.int32)

    @pl.when(s < NS - 1)
    def _scan_a():
        pltpu.make_async_copy(sid.at[pl.ds(0, SPS)], sbuf, psem).wait()
        cv = count_groups(SPS // 16, cvec)
        for j in range(8):
            cnt_v[pl.ds(16 * j, 16)] = cv

    @pl.when(s == NS - 1)
    def _scan_b():
        pltpu.make_async_copy(sid.at[pl.ds(0, SPL)], sbuf.at[pl.ds(0, SPL)],
                              psem).wait()
        pltpu.make_async_copy(sid.at[pl.ds(0, TAIL)],
                              sbuf.at[pl.ds(0, TAIL)], psem).wait()
        cv = count_groups((SPL + TAIL) // 16, cvec)
        for j in range(8):
            cnt_v[pl.ds(16 * j, 16)] = cv

    pltpu.sync_copy(cnt_v, cnt_sh.at[s])
    plsc.subcore_barrier()
    pltpu.sync_copy(cnt_sh, cmat)

    def addrow(i, tv):
        return tv + cmat[i, pl.ds(0, 16)]
    tot = lax.fori_loop(0, NS, addrow, jnp.zeros((16,), jnp.int32))
    r_star = jnp.sum(tot)

    for _ in range(2):
        pltpu.make_async_copy(ne.at[gidx.at[0]], rows.at[0], gsem).wait()
    for h in range(2):
        pltpu.sync_copy(rows.at[h],
                        out.at[pl.ds(base + h * 128, 128), pl.ds(D, D)])

    nc0 = jnp.minimum((r_star + CH - 1) // CH, NCH)
    k0 = jnp.minimum(r_star // CH, NCH)
    my_lo = jnp.where(c == 0, s, k0 + s)
    my_hi = jnp.where(c == 0, nc0, NCH)
    nit = jnp.maximum(my_hi - my_lo + NS - 1, 0) // NS

    def fire(i, slot):
        start = pl.multiple_of((my_lo + i * NS) * CH, CH)
        pltpu.async_copy(sid.at[pl.ds(start, CH)], ids.at[slot, 0], isem)
        pltpu.async_copy(ne.at[pl.ds(start, CH)], rows.at[slot], rsem)

    @pl.when(nit > 0)
    def _prime():
        fire(0, 0)

    def it(g, _):
        @pl.when(g + 1 < nit)
        def _prefetch():
            fire(g + 1, lax.rem(g + 1, 2))
        pltpu.make_async_copy(sid.at[pl.ds(0, CH)], ids.at[0, 0],
                              isem).wait()
        pltpu.make_async_copy(ne.at[pl.ds(0, CH)], rows.at[0], rsem).wait()
        slot = lax.rem(g, 2)
        pltpu.sync_copy(rows.at[slot], acc.at[ids.at[slot, 0]], add=True)
        return 0

    lax.fori_loop(0, nit, it, 0)

    @pl.when(s == NS - 1)
    def _tail():
        pltpu.sync_copy(sid.at[pl.ds(NCH * CH, TAIL)], idt.at[0])
        pltpu.sync_copy(ne.at[pl.ds(NCH * CH, TAIL)], rowst)
        pltpu.sync_copy(rowst, acc.at[idt.at[0]], add=True)

    for h in range(2):
        pltpu.async_copy(ne.at[gidx.at[2 + h]], rows.at[h], gsem)
    for _ in range(2):
        pltpu.make_async_copy(ne.at[gidx.at[0]], rows.at[0], gsem).wait()
    for h in range(2):
        pltpu.sync_copy(rows.at[h],
                        out.at[pl.ds(base + h * 128, 128), pl.ds(2 * D, D)])

    plsc.subcore_barrier()

    obase = pl.multiple_of(c * CUT + s * GPW, GPW)
    for h in range(2):
        pltpu.sync_copy(acc.at[pl.ds(obase + h * 128, 128)], rows.at[h])
        pltpu.sync_copy(rows.at[h],
                        out.at[pl.ds(obase + h * 128, 128), pl.ds(0, D)])


_sc_call = pl.kernel(
    _body,
    out_type=jax.ShapeDtypeStruct((B, 3 * D), jnp.float32),
    mesh=plsc.VectorSubcoreMesh(core_axis_name="c", subcore_axis_name="s"),
    scratch_types=[
        pltpu.VMEM((4, 128), jnp.int32),
        pltpu.VMEM((2, 1, CH), jnp.int32),
        pltpu.VMEM((2, CH, D), jnp.float32),
        pltpu.VMEM((SPS,), jnp.int32),
        pltpu.VMEM((1, TAIL), jnp.int32),
        pltpu.VMEM((TAIL, D), jnp.float32),
        pltpu.VMEM((128,), jnp.int32),
        pltpu.VMEM((NS, 128), jnp.int32),
        pltpu.VMEM_SHARED((B, D), jnp.float32),
        pltpu.VMEM_SHARED((NS, 128), jnp.int32),
        pltpu.SemaphoreType.DMA,
        pltpu.SemaphoreType.DMA,
        pltpu.SemaphoreType.DMA,
        pltpu.SemaphoreType.DMA,
    ],
    compiler_params=pltpu.CompilerParams(use_tc_tiling_on_sc=True,
                                         needs_layout_passes=False),
)


def kernel(node_embed, segment_ids, start_idx, end_idx):
    return _sc_call(node_embed,
                    segment_ids.astype(jnp.int32),
                    start_idx.astype(jnp.int32),
                    end_idx.astype(jnp.int32))


# speedup vs baseline: 6.1396x; 1.0061x over previous
import jax
import jax.numpy as jnp
from jax import lax
from jax.experimental import pallas as pl
from jax.experimental.pallas import tpu as pltpu
from jax.experimental.pallas import tpu_sc as plsc

N = 100000
D = 128
B = 8192
NC = 2
NS = 16
CUT = B // 2
GPW = B // (NC * NS)
CH = 128
NCH = N // CH
TAIL = N - NCH * CH
SPS = 6272
SPL = N - (NS - 1) * SPS


def _body(ne, sid, st, en, out, gidx, ids, rows, sbuf, idt, rowst,
          cnt_v, cmat, acc, cnt_sh, gsem, isem, rsem, psem):
    c = lax.axis_index("c")
    s = lax.axis_index("s")
    w = c * NS + s

    soff = pl.multiple_of(s * SPS, 8)

    @pl.when(s < NS - 1)
    def _ps_a():
        pltpu.async_copy(sid.at[pl.ds(soff, SPS)], sbuf, psem)

    @pl.when(s == NS - 1)
    def _ps_b():
        pltpu.async_copy(sid.at[pl.ds(soff, SPL)], sbuf.at[pl.ds(0, SPL)],
                         psem)
        pltpu.async_copy(sid.at[pl.ds(NCH * CH, TAIL)],
                         sbuf.at[pl.ds(SPL, TAIL)], psem)

    base = pl.multiple_of(w * GPW, GPW)
    for h in range(2):
        pltpu.async_copy(st.at[pl.ds(base + h * 128, 128)], gidx.at[h], isem)
        pltpu.async_copy(en.at[pl.ds(base + h * 128, 128)], gidx.at[2 + h],
                         isem)
    for _ in range(4):
        pltpu.make_async_copy(st.at[pl.ds(0, 128)], gidx.at[0], isem).wait()
    for h in range(2):
        pltpu.async_copy(ne.at[gidx.at[h]], rows.at[h], gsem)

    def zero_step(i, _):
        rowst[i // 8, pl.ds((i % 8) * 16, 16)] = jnp.zeros((16,), jnp.float32)
        return 0
    lax.fori_loop(0, TAIL * 8, zero_step, 0)
    zbase = pl.multiple_of(s * 512, 512)
    for k in range(512 // TAIL):
        pltpu.sync_copy(rowst, acc.at[pl.ds(zbase + k * TAIL, TAIL)])

    def count_groups(ngroups, cvec):
        def ld(i, cv):
            v = sbuf[pl.ds(i * 16, 16)]
            return cv + jnp.where(v < CUT, 1, 0).astype(jnp.int32)
        return lax.fori_loop(0, ngroups, ld, cvec)

    cvec = jnp.zeros((16,), jnp.int32)

    @pl.when(s < NS - 1)
    def _scan_a():
        pltpu.make_async_copy(sid.at[pl.ds(0, SPS)], sbuf, psem).wait()
        cv = count_groups(SPS // 16, cvec)
        for j in range(8):
            cnt_v[pl.ds(16 * j, 16)] = cv

    @pl.when(s == NS - 1)
    def _scan_b():
        pltpu.make_async_copy(sid.at[pl.ds(0, SPL)], sbuf.at[pl.ds(0, SPL)],
                              psem).wait()
        pltpu.make_async_copy(sid.at[pl.ds(0, TAIL)],
                              sbuf.at[pl.ds(0, TAIL)], psem).wait()
        cv = count_groups((SPL + TAIL) // 16, cvec)
        for j in range(8):
            cnt_v[pl.ds(16 * j, 16)] = cv

    pltpu.sync_copy(cnt_v, cnt_sh.at[s])
    plsc.subcore_barrier()
    pltpu.sync_copy(cnt_sh, cmat)

    def addrow(i, tv):
        return tv + cmat[i, pl.ds(0, 16)]
    tot = lax.fori_loop(0, NS, addrow, jnp.zeros((16,), jnp.int32))
    r_star = jnp.sum(tot)

    for _ in range(2):
        pltpu.make_async_copy(ne.at[gidx.at[0]], rows.at[0], gsem).wait()
    for h in range(2):
        pltpu.async_copy(rows.at[h],
                         out.at[pl.ds(base + h * 128, 128), pl.ds(D, D)],
                         gsem)

    nc0 = jnp.minimum((r_star + CH - 1) // CH, NCH)
    k0 = jnp.minimum(r_star // CH, NCH)
    my_lo = jnp.where(c == 0, s, k0 + s)
    my_hi = jnp.where(c == 0, nc0, NCH)
    nit = jnp.maximum(my_hi - my_lo + NS - 1, 0) // NS

    for h in range(2):
        pltpu.make_async_copy(rows.at[h],
                              out.at[pl.ds(base, 128), pl.ds(D, D)],
                              gsem).wait()

    def fire(i, slot):
        start = pl.multiple_of((my_lo + i * NS) * CH, CH)
        pltpu.async_copy(sid.at[pl.ds(start, CH)], ids.at[slot, 0], isem)
        pltpu.async_copy(ne.at[pl.ds(start, CH)], rows.at[slot], rsem)

    @pl.when(nit > 0)
    def _prime():
        fire(0, 0)

    def it(g, _):
        @pl.when(g + 1 < nit)
        def _prefetch():
            fire(g + 1, lax.rem(g + 1, 2))
        pltpu.make_async_copy(sid.at[pl.ds(0, CH)], ids.at[0, 0],
                              isem).wait()
        pltpu.make_async_copy(ne.at[pl.ds(0, CH)], rows.at[0], rsem).wait()
        slot = lax.rem(g, 2)
        pltpu.sync_copy(rows.at[slot], acc.at[ids.at[slot, 0]], add=True)
        return 0

    lax.fori_loop(0, nit, it, 0)

    @pl.when(s == NS - 1)
    def _tail():
        pltpu.sync_copy(sid.at[pl.ds(NCH * CH, TAIL)], idt.at[0])
        pltpu.sync_copy(ne.at[pl.ds(NCH * CH, TAIL)], rowst)
        pltpu.sync_copy(rowst, acc.at[idt.at[0]], add=True)

    for h in range(2):
        pltpu.async_copy(ne.at[gidx.at[2 + h]], rows.at[h], gsem)
    for _ in range(2):
        pltpu.make_async_copy(ne.at[gidx.at[0]], rows.at[0], gsem).wait()
    for h in range(2):
        pltpu.sync_copy(rows.at[h],
                        out.at[pl.ds(base + h * 128, 128), pl.ds(2 * D, D)])

    plsc.subcore_barrier()

    obase = pl.multiple_of(c * CUT + s * GPW, GPW)
    pltpu.async_copy(acc.at[pl.ds(obase, 128)], rows.at[0], rsem)
    pltpu.make_async_copy(acc.at[pl.ds(obase, 128)], rows.at[0], rsem).wait()
    pltpu.async_copy(acc.at[pl.ds(obase + 128, 128)], rows.at[1], rsem)
    pltpu.async_copy(rows.at[0],
                     out.at[pl.ds(obase, 128), pl.ds(0, D)], gsem)
    pltpu.make_async_copy(acc.at[pl.ds(obase, 128)], rows.at[1], rsem).wait()
    pltpu.async_copy(rows.at[1],
                     out.at[pl.ds(obase + 128, 128), pl.ds(0, D)], gsem)
    for _ in range(2):
        pltpu.make_async_copy(rows.at[0],
                              out.at[pl.ds(obase, 128), pl.ds(0, D)],
                              gsem).wait()


_sc_call = pl.kernel(
    _body,
    out_type=jax.ShapeDtypeStruct((B, 3 * D), jnp.float32),
    mesh=plsc.VectorSubcoreMesh(core_axis_name="c", subcore_axis_name="s"),
    scratch_types=[
        pltpu.VMEM((4, 128), jnp.int32),
        pltpu.VMEM((2, 1, CH), jnp.int32),
        pltpu.VMEM((2, CH, D), jnp.float32),
        pltpu.VMEM((SPS,), jnp.int32),
        pltpu.VMEM((1, TAIL), jnp.int32),
        pltpu.VMEM((TAIL, D), jnp.float32),
        pltpu.VMEM((128,), jnp.int32),
        pltpu.VMEM((NS, 128), jnp.int32),
        pltpu.VMEM_SHARED((B, D), jnp.float32),
        pltpu.VMEM_SHARED((NS, 128), jnp.int32),
        pltpu.SemaphoreType.DMA,
        pltpu.SemaphoreType.DMA,
        pltpu.SemaphoreType.DMA,
        pltpu.SemaphoreType.DMA,
    ],
    compiler_params=pltpu.CompilerParams(use_tc_tiling_on_sc=True,
                                         needs_layout_passes=False),
)


def kernel(node_embed, segment_ids, start_idx, end_idx):
    return _sc_call(node_embed,
                    segment_ids.astype(jnp.int32),
                    start_idx.astype(jnp.int32),
                    end_idx.astype(jnp.int32))
